# baseline (speedup 1.0000x reference)
import dataclasses
import functools

import jax
import jax.numpy as jnp
from jax import lax
from jax.experimental import pallas as pl
from jax.experimental.pallas import tpu as pltpu
from jax.experimental.pallas import tpu_sc as plsc

NC = 2
NS = 16
NW = NC * NS
CH = 104
NP = 10240
_BN = 1280


def _sc_mesh():
    return plsc.VectorSubcoreMesh(core_axis_name="c", subcore_axis_name="s")


def _sc_compiler_params():
    cp = pltpu.CompilerParams()
    if "needs_layout_passes" in pltpu.CompilerParams.__dataclass_fields__:
        cp = dataclasses.replace(cp, needs_layout_passes=False)
    return cp


def _degrees_sc(src, dst, zeros1d, epw):
    @functools.partial(
        pl.kernel,
        out_type=[
            jax.ShapeDtypeStruct((NW, NP), jnp.float32),
            jax.ShapeDtypeStruct((NW, NP), jnp.float32),
        ],
        mesh=_sc_mesh(),
        compiler_params=_sc_compiler_params(),
        scratch_types=[
            pltpu.VMEM((epw,), jnp.int32),
            pltpu.VMEM((epw,), jnp.int32),
            pltpu.VMEM((NP,), jnp.float32),
            pltpu.VMEM((NP,), jnp.float32),
        ],
    )
    def deg_kernel(src_hbm, dst_hbm, z_hbm, out_s_hbm, out_d_hbm,
                   idx_s, idx_d, tab_s, tab_d):
        cid = lax.axis_index("c")
        sid = lax.axis_index("s")
        pltpu.sync_copy(z_hbm, tab_s)
        pltpu.sync_copy(z_hbm, tab_d)
        wid = cid * NS + sid
        pltpu.sync_copy(src_hbm.at[pl.ds(wid * epw, epw)], idx_s)
        pltpu.sync_copy(dst_hbm.at[pl.ds(wid * epw, epw)], idx_d)
        ones_vec = jnp.full((16,), 1.0, jnp.float32)

        @pl.loop(0, epw // 16)
        def _(j):
            iv_s = idx_s[pl.ds(j * 16, 16)]
            iv_d = idx_d[pl.ds(j * 16, 16)]
            plsc.addupdate_scatter(tab_s, [iv_s], ones_vec)
            plsc.addupdate_scatter(tab_d, [iv_d], ones_vec)

        pltpu.sync_copy(tab_s, out_s_hbm.at[wid])
        pltpu.sync_copy(tab_d, out_d_hbm.at[wid])

    return deg_kernel(src, dst, zeros1d)


def _agg_sc(v, src, dst, zeros, epw):
    d = v.shape[1]
    cpw = epw // CH
    rps = NP // NS
    npairs = (cpw - 1) // 2

    @functools.partial(
        pl.kernel,
        out_type=jax.ShapeDtypeStruct((NC, NP, d), jnp.float32),
        mesh=_sc_mesh(),
        scratch_types=[
            pltpu.VMEM((CH,), jnp.int32),
            pltpu.VMEM((CH,), jnp.int32),
            pltpu.VMEM((CH,), jnp.int32),
            pltpu.VMEM((CH,), jnp.int32),
            pltpu.VMEM((CH, d), jnp.float32),
            pltpu.VMEM((CH, d), jnp.float32),
            pltpu.VMEM_SHARED((NP, d), jnp.float32),
            pltpu.SemaphoreType.DMA,
            pltpu.SemaphoreType.DMA,
            pltpu.SemaphoreType.DMA,
            pltpu.SemaphoreType.DMA,
        ],
    )
    def agg_kernel(v_hbm, src_hbm, dst_hbm, z_hbm, out_hbm,
                   idx_s0, idx_d0, idx_s1, idx_d1, rows0, rows1, tab,
                   semg0, semg1, semi0, semi1):
        cid = lax.axis_index("c")
        sid = lax.axis_index("s")
        pltpu.sync_copy(z_hbm.at[pl.ds(sid * rps, rps)],
                        tab.at[pl.ds(sid * rps, rps)])
        wid = cid * NS + sid
        base = wid * epw
        plsc.subcore_barrier()

        def idx_load(c, idx_s, idx_d, sem):
            pltpu.async_copy(src_hbm.at[pl.ds(base + c * CH, CH)], idx_s, sem)
            pltpu.async_copy(dst_hbm.at[pl.ds(base + c * CH, CH)], idx_d, sem)

        def idx_wait(idx_s, idx_d, sem):
            pltpu.make_async_copy(src_hbm.at[pl.ds(base, CH)], idx_s,
                                  sem).wait()
            pltpu.make_async_copy(dst_hbm.at[pl.ds(base, CH)], idx_d,
                                  sem).wait()

        idx_load(0, idx_s0, idx_d0, semi0)
        idx_wait(idx_s0, idx_d0, semi0)
        pltpu.async_copy(v_hbm.at[idx_s0], rows0, semg0)
        idx_load(1, idx_s1, idx_d1, semi1)

        @pl.loop(0, npairs)
        def _(t):
            c1 = t * 2 + 1
            idx_wait(idx_s1, idx_d1, semi1)
            pltpu.async_copy(v_hbm.at[idx_s1], rows1, semg1)
            pltpu.make_async_copy(v_hbm.at[idx_s0], rows0, semg0).wait()
            pltpu.sync_copy(rows0, tab.at[idx_d0], add=True)
            idx_load(c1 + 1, idx_s0, idx_d0, semi0)
            idx_wait(idx_s0, idx_d0, semi0)
            pltpu.async_copy(v_hbm.at[idx_s0], rows0, semg0)
            pltpu.make_async_copy(v_hbm.at[idx_s1], rows1, semg1).wait()
            pltpu.sync_copy(rows1, tab.at[idx_d1], add=True)
            pltpu.async_copy(src_hbm.at[pl.ds(base + (c1 + 2) * CH, CH)],
                             idx_s1, semi1)
            pltpu.async_copy(dst_hbm.at[pl.ds(base + (c1 + 2) * CH, CH)],
                             idx_d1, semi1)

        pltpu.make_async_copy(v_hbm.at[idx_s0], rows0, semg0).wait()
        pltpu.sync_copy(rows0, tab.at[idx_d0], add=True)
        idx_wait(idx_s1, idx_d1, semi1)

        plsc.subcore_barrier()
        pltpu.sync_copy(tab.at[pl.ds(sid * rps, rps)],
                        out_hbm.at[cid, pl.ds(sid * rps, rps)])

    return agg_kernel(v, src, dst, zeros)


def _prep_tc(od_parts, id_parts, xp):
    d = xp.shape[1]

    def body(od_ref, id_ref, x_ref, ns_ref, nd_ref, v_ref):
        od = jnp.sum(od_ref[...], axis=0)
        idg = jnp.sum(id_ref[...], axis=0)
        ns = lax.rsqrt(jnp.maximum(od, 1.0))
        nd = lax.rsqrt(jnp.maximum(idg, 1.0))
        ns_ref[...] = jnp.broadcast_to(ns[:, None], (_BN, 16))
        nd_ref[...] = jnp.broadcast_to(nd[:, None], (_BN, 16))
        v_ref[...] = x_ref[...] * ns[:, None]

    return pl.pallas_call(
        body,
        grid=(NP // _BN,),
        in_specs=[
            pl.BlockSpec((NW, _BN), lambda i: (0, i)),
            pl.BlockSpec((NW, _BN), lambda i: (0, i)),
            pl.BlockSpec((_BN, d), lambda i: (i, 0)),
        ],
        out_specs=[
            pl.BlockSpec((_BN, 16), lambda i: (i, 0)),
            pl.BlockSpec((_BN, 16), lambda i: (i, 0)),
            pl.BlockSpec((_BN, d), lambda i: (i, 0)),
        ],
        out_shape=[
            jax.ShapeDtypeStruct((NP, 16), jnp.float32),
            jax.ShapeDtypeStruct((NP, 16), jnp.float32),
            jax.ShapeDtypeStruct((NP, d), jnp.float32),
        ],
    )(od_parts, id_parts, xp)


def _layer_tc(t_parts, nd, ns, xp, W, b):
    d = xp.shape[1]

    def body(t_ref, nd_ref, ns_ref, x_ref, w_ref, b_ref, o_ref):
        t = (t_ref[0] + t_ref[1]) * nd_ref[:, :1]
        h = lax.dot_general(t, w_ref[...], (((1,), (0,)), ((), ())),
                            precision=lax.Precision.HIGHEST)
        h = jnp.maximum(h + b_ref[...], 0.0) + x_ref[...]
        o_ref[...] = h * ns_ref[:, :1]

    return pl.pallas_call(
        body,
        grid=(NP // _BN,),
        in_specs=[
            pl.BlockSpec((NC, _BN, d), lambda i: (0, i, 0)),
            pl.BlockSpec((_BN, 16), lambda i: (i, 0)),
            pl.BlockSpec((_BN, 16), lambda i: (i, 0)),
            pl.BlockSpec((_BN, d), lambda i: (i, 0)),
            pl.BlockSpec((d, d), lambda i: (0, 0)),
            pl.BlockSpec((1, d), lambda i: (0, 0)),
        ],
        out_specs=pl.BlockSpec((_BN, d), lambda i: (i, 0)),
        out_shape=jax.ShapeDtypeStruct((NP, d), jnp.float32),
    )(t_parts, nd, ns, xp, W, b)


def _final_tc(t_parts, nd, W4, b4, Wfc, bfc):
    d = W4.shape[0]
    c = Wfc.shape[1]

    def body(t_ref, nd_ref, w4_ref, b4_ref, wfc_ref, bfc_ref, o_ref):
        t = (t_ref[0] + t_ref[1]) * nd_ref[:, :1]
        h = lax.dot_general(t, w4_ref[...], (((1,), (0,)), ((), ())),
                            precision=lax.Precision.HIGHEST)
        h = jnp.maximum(h + b4_ref[...], 0.0)
        o_ref[...] = lax.dot_general(h, wfc_ref[...], (((1,), (0,)), ((), ())),
                                     precision=lax.Precision.HIGHEST) + bfc_ref[...]

    return pl.pallas_call(
        body,
        grid=(NP // _BN,),
        in_specs=[
            pl.BlockSpec((NC, _BN, d), lambda i: (0, i, 0)),
            pl.BlockSpec((_BN, 16), lambda i: (i, 0)),
            pl.BlockSpec((d, d), lambda i: (0, 0)),
            pl.BlockSpec((1, d), lambda i: (0, 0)),
            pl.BlockSpec((d, c), lambda i: (0, 0)),
            pl.BlockSpec((1, c), lambda i: (0, 0)),
        ],
        out_specs=pl.BlockSpec((_BN, c), lambda i: (i, 0)),
        out_shape=jax.ShapeDtypeStruct((NP, c), jnp.float32),
    )(t_parts, nd, W4, b4, Wfc, bfc)


def kernel(x, edge_index, W1, b1, W2, b2, W3, b3, W4, b4, Wfc, bfc):
    n, d = x.shape
    e = edge_index.shape[1]
    c = Wfc.shape[1]
    cpw_pad = -(-e // (NW * CH))
    if cpw_pad % 2 == 0:
        cpw_pad += 1
    ep = cpw_pad * CH * NW
    epw_pad = cpw_pad * CH
    padv = jnp.full((ep - e + CH,), NP - 1, jnp.int32)
    src = jnp.concatenate([edge_index[0], padv])
    dst = jnp.concatenate([edge_index[1], padv])
    zeros = jnp.zeros((NP, d), jnp.float32)
    zeros1d = jnp.zeros((NP,), jnp.float32)
    xp = jnp.pad(x, ((0, NP - n), (0, 0)))

    od_parts, id_parts = _degrees_sc(src, dst, zeros1d, epw_pad)
    ns, nd, v = _prep_tc(od_parts, id_parts, xp)
    for Wk, bk in ((W1, b1), (W2, b2), (W3, b3)):
        t_parts = _agg_sc(v, src, dst, zeros, epw_pad)
        v = _layer_tc(t_parts, nd, ns, xp, Wk, bk.reshape(1, d))
    t_parts = _agg_sc(v, src, dst, zeros, epw_pad)
    y = _final_tc(t_parts, nd, W4, b4.reshape(1, d), Wfc, bfc.reshape(1, c))
    return y[:n]


# speedup vs baseline: 1.7591x; 1.7591x over previous
import dataclasses
import functools

import jax
import jax.numpy as jnp
from jax import lax
from jax.experimental import pallas as pl
from jax.experimental.pallas import tpu as pltpu
from jax.experimental.pallas import tpu_sc as plsc

NC = 2
NS = 16
NW = NC * NS
CH = 80
NP = 10240
_BN = 1280


def _sc_mesh():
    return plsc.VectorSubcoreMesh(core_axis_name="c", subcore_axis_name="s")


def _sc_compiler_params():
    cp = pltpu.CompilerParams()
    if "needs_layout_passes" in pltpu.CompilerParams.__dataclass_fields__:
        cp = dataclasses.replace(cp, needs_layout_passes=False)
    return cp


def _degrees_sc(src, dst, zeros1d, epw):
    @functools.partial(
        pl.kernel,
        out_type=[
            jax.ShapeDtypeStruct((NW, NP), jnp.float32),
            jax.ShapeDtypeStruct((NW, NP), jnp.float32),
        ],
        mesh=_sc_mesh(),
        compiler_params=_sc_compiler_params(),
        scratch_types=[
            pltpu.VMEM((epw,), jnp.int32),
            pltpu.VMEM((epw,), jnp.int32),
            pltpu.VMEM((NP,), jnp.float32),
            pltpu.VMEM((NP,), jnp.float32),
        ],
    )
    def deg_kernel(src_hbm, dst_hbm, z_hbm, out_s_hbm, out_d_hbm,
                   idx_s, idx_d, tab_s, tab_d):
        cid = lax.axis_index("c")
        sid = lax.axis_index("s")
        pltpu.sync_copy(z_hbm, tab_s)
        pltpu.sync_copy(z_hbm, tab_d)
        wid = cid * NS + sid
        pltpu.sync_copy(src_hbm.at[pl.ds(wid * epw, epw)], idx_s)
        pltpu.sync_copy(dst_hbm.at[pl.ds(wid * epw, epw)], idx_d)
        ones_vec = jnp.full((16,), 1.0, jnp.float32)

        @pl.loop(0, epw // 16)
        def _(j):
            iv_s = idx_s[pl.ds(j * 16, 16)]
            iv_d = idx_d[pl.ds(j * 16, 16)]
            plsc.addupdate_scatter(tab_s, [iv_s], ones_vec)
            plsc.addupdate_scatter(tab_d, [iv_d], ones_vec)

        pltpu.sync_copy(tab_s, out_s_hbm.at[wid])
        pltpu.sync_copy(tab_d, out_d_hbm.at[wid])

    return deg_kernel(src, dst, zeros1d)


def _agg_sc(v, src, dst, zeros, epw):
    d = v.shape[1]
    cpw = epw // CH
    rps = NP // NS
    npairs = (cpw - 1) // 2

    @functools.partial(
        pl.kernel,
        out_type=jax.ShapeDtypeStruct((NC, NP, d), jnp.float32),
        mesh=_sc_mesh(),
        scratch_types=[
            pltpu.VMEM((CH,), jnp.int32),
            pltpu.VMEM((CH,), jnp.int32),
            pltpu.VMEM((CH,), jnp.int32),
            pltpu.VMEM((CH,), jnp.int32),
            pltpu.VMEM((CH, d), jnp.float32),
            pltpu.VMEM((CH, d), jnp.float32),
            pltpu.VMEM_SHARED((NP, d), jnp.float32),
            pltpu.SemaphoreType.DMA,
            pltpu.SemaphoreType.DMA,
            pltpu.SemaphoreType.DMA,
            pltpu.SemaphoreType.DMA,
        ],
    )
    def agg_kernel(v_hbm, src_hbm, dst_hbm, z_hbm, out_hbm,
                   idx_s0, idx_d0, idx_s1, idx_d1, rows0, rows1, tab,
                   semg0, semg1, semi0, semi1):
        cid = lax.axis_index("c")
        sid = lax.axis_index("s")
        pltpu.sync_copy(z_hbm.at[pl.ds(sid * rps, rps)],
                        tab.at[pl.ds(sid * rps, rps)])
        wid = cid * NS + sid
        base = wid * epw
        plsc.subcore_barrier()

        def idx_load(c, idx_s, idx_d, sem):
            pltpu.async_copy(src_hbm.at[pl.ds(base + c * CH, CH)], idx_s, sem)
            pltpu.async_copy(dst_hbm.at[pl.ds(base + c * CH, CH)], idx_d, sem)

        def idx_wait(idx_s, idx_d, sem):
            pltpu.make_async_copy(src_hbm.at[pl.ds(base, CH)], idx_s,
                                  sem).wait()
            pltpu.make_async_copy(dst_hbm.at[pl.ds(base, CH)], idx_d,
                                  sem).wait()

        idx_load(0, idx_s0, idx_d0, semi0)
        idx_wait(idx_s0, idx_d0, semi0)
        pltpu.async_copy(v_hbm.at[idx_s0], rows0, semg0)
        idx_load(1, idx_s1, idx_d1, semi1)

        @pl.loop(0, npairs)
        def _(t):
            c1 = t * 2 + 1
            idx_wait(idx_s1, idx_d1, semi1)
            pltpu.async_copy(v_hbm.at[idx_s1], rows1, semg1)
            pltpu.make_async_copy(v_hbm.at[idx_s0], rows0, semg0).wait()
            pltpu.sync_copy(rows0, tab.at[idx_d0], add=True)
            idx_load(c1 + 1, idx_s0, idx_d0, semi0)
            idx_wait(idx_s0, idx_d0, semi0)
            pltpu.async_copy(v_hbm.at[idx_s0], rows0, semg0)
            pltpu.make_async_copy(v_hbm.at[idx_s1], rows1, semg1).wait()
            pltpu.sync_copy(rows1, tab.at[idx_d1], add=True)
            pltpu.async_copy(src_hbm.at[pl.ds(base + (c1 + 2) * CH, CH)],
                             idx_s1, semi1)
            pltpu.async_copy(dst_hbm.at[pl.ds(base + (c1 + 2) * CH, CH)],
                             idx_d1, semi1)

        pltpu.make_async_copy(v_hbm.at[idx_s0], rows0, semg0).wait()
        pltpu.sync_copy(rows0, tab.at[idx_d0], add=True)
        idx_wait(idx_s1, idx_d1, semi1)

        plsc.subcore_barrier()
        pltpu.sync_copy(tab.at[pl.ds(sid * rps, rps)],
                        out_hbm.at[cid, pl.ds(sid * rps, rps)])

    return agg_kernel(v, src, dst, zeros)


def _prep_tc(od_parts, id_parts, xp):
    d = xp.shape[1]

    def body(od_ref, id_ref, x_ref, ns_ref, nd_ref, v_ref):
        od = jnp.sum(od_ref[...], axis=0)
        idg = jnp.sum(id_ref[...], axis=0)
        ns = lax.rsqrt(jnp.maximum(od, 1.0))
        nd = lax.rsqrt(jnp.maximum(idg, 1.0))
        ns_ref[...] = jnp.broadcast_to(ns[:, None], (_BN, 16))
        nd_ref[...] = jnp.broadcast_to(nd[:, None], (_BN, 16))
        v_ref[...] = x_ref[...] * ns[:, None]

    return pl.pallas_call(
        body,
        grid=(NP // _BN,),
        in_specs=[
            pl.BlockSpec((NW, _BN), lambda i: (0, i)),
            pl.BlockSpec((NW, _BN), lambda i: (0, i)),
            pl.BlockSpec((_BN, d), lambda i: (i, 0)),
        ],
        out_specs=[
            pl.BlockSpec((_BN, 16), lambda i: (i, 0)),
            pl.BlockSpec((_BN, 16), lambda i: (i, 0)),
            pl.BlockSpec((_BN, d), lambda i: (i, 0)),
        ],
        out_shape=[
            jax.ShapeDtypeStruct((NP, 16), jnp.float32),
            jax.ShapeDtypeStruct((NP, 16), jnp.float32),
            jax.ShapeDtypeStruct((NP, d), jnp.float32),
        ],
    )(od_parts, id_parts, xp)


def _layer_tc(t_parts, nd, ns, xp, W, b):
    d = xp.shape[1]

    def body(t_ref, nd_ref, ns_ref, x_ref, w_ref, b_ref, o_ref):
        t = (t_ref[0] + t_ref[1]) * nd_ref[:, :1]
        h = lax.dot_general(t, w_ref[...], (((1,), (0,)), ((), ())),
                            precision=lax.Precision.HIGHEST)
        h = jnp.maximum(h + b_ref[...], 0.0) + x_ref[...]
        o_ref[...] = h * ns_ref[:, :1]

    return pl.pallas_call(
        body,
        grid=(NP // _BN,),
        in_specs=[
            pl.BlockSpec((NC, _BN, d), lambda i: (0, i, 0)),
            pl.BlockSpec((_BN, 16), lambda i: (i, 0)),
            pl.BlockSpec((_BN, 16), lambda i: (i, 0)),
            pl.BlockSpec((_BN, d), lambda i: (i, 0)),
            pl.BlockSpec((d, d), lambda i: (0, 0)),
            pl.BlockSpec((1, d), lambda i: (0, 0)),
        ],
        out_specs=pl.BlockSpec((_BN, d), lambda i: (i, 0)),
        out_shape=jax.ShapeDtypeStruct((NP, d), jnp.float32),
    )(t_parts, nd, ns, xp, W, b)


def _final_tc(t_parts, nd, W4, b4, Wfc, bfc):
    d = W4.shape[0]
    c = Wfc.shape[1]

    def body(t_ref, nd_ref, w4_ref, b4_ref, wfc_ref, bfc_ref, o_ref):
        t = (t_ref[0] + t_ref[1]) * nd_ref[:, :1]
        h = lax.dot_general(t, w4_ref[...], (((1,), (0,)), ((), ())),
                            precision=lax.Precision.HIGHEST)
        h = jnp.maximum(h + b4_ref[...], 0.0)
        o_ref[...] = lax.dot_general(h, wfc_ref[...], (((1,), (0,)), ((), ())),
                                     precision=lax.Precision.HIGHEST) + bfc_ref[...]

    return pl.pallas_call(
        body,
        grid=(NP // _BN,),
        in_specs=[
            pl.BlockSpec((NC, _BN, d), lambda i: (0, i, 0)),
            pl.BlockSpec((_BN, 16), lambda i: (i, 0)),
            pl.BlockSpec((d, d), lambda i: (0, 0)),
            pl.BlockSpec((1, d), lambda i: (0, 0)),
            pl.BlockSpec((d, c), lambda i: (0, 0)),
            pl.BlockSpec((1, c), lambda i: (0, 0)),
        ],
        out_specs=pl.BlockSpec((_BN, c), lambda i: (i, 0)),
        out_shape=jax.ShapeDtypeStruct((NP, c), jnp.float32),
    )(t_parts, nd, W4, b4, Wfc, bfc)


def kernel(x, edge_index, W1, b1, W2, b2, W3, b3, W4, b4, Wfc, bfc):
    n, d = x.shape
    e = edge_index.shape[1]
    c = Wfc.shape[1]
    cpw_pad = -(-e // (NW * CH))
    if cpw_pad % 2 == 0:
        cpw_pad += 1
    ep = cpw_pad * CH * NW
    epw_pad = cpw_pad * CH
    padv = jnp.full((ep - e + CH,), NP - 1, jnp.int32)
    src = jnp.concatenate([edge_index[0], padv])
    dst = jnp.concatenate([edge_index[1], padv])
    zeros = jnp.zeros((NP, d), jnp.float32)
    zeros1d = jnp.zeros((NP,), jnp.float32)
    xp = jnp.pad(x, ((0, NP - n), (0, 0)))

    od_parts, id_parts = _degrees_sc(src, dst, zeros1d, epw_pad)
    ns, nd, v = _prep_tc(od_parts, id_parts, xp)
    for Wk, bk in ((W1, b1), (W2, b2), (W3, b3)):
        t_parts = _agg_sc(v, src, dst, zeros, epw_pad)
        v = _layer_tc(t_parts, nd, ns, xp, Wk, bk.reshape(1, d))
    t_parts = _agg_sc(v, src, dst, zeros, epw_pad)
    y = _final_tc(t_parts, nd, W4, b4.reshape(1, d), Wfc, bfc.reshape(1, c))
    return y[:n]
